# baseline (speedup 1.0000x reference)
import functools
import math

import jax
import jax.numpy as jnp
from jax import lax
from jax.experimental import pallas as pl
from jax.experimental.pallas import tpu as pltpu
from jax.experimental.pallas import tpu_sc as plsc

_Z_DIM = 10
_M = 4096
_N = 1024
_NC = 2
_NS = 16
_L = 16
_NW = _NC * _NS
_RPW = _M // _NW
_CH = 16


def _affine_consts():
    p = 0.75
    zs = []
    for k in range(_Z_DIM):
        geo = k * math.log(1.0 - p) + math.log(p)
        log_comb = (
            math.lgamma(_Z_DIM + 1.0)
            - math.lgamma(k + 1.0)
            - math.lgamma(_Z_DIM - k + 1.0)
        )
        zs.append(log_comb + geo)
    mx = max(zs)
    z = mx + math.log(sum(math.exp(v - mx) for v in zs))
    a = math.log(1.0 - p)
    b = math.log(p) - z
    return a, b


_A, _B = _affine_consts()


def _sc_body(zt_rep_hbm, zmt_hbm, out_hbm, zm_v, zt_v, out_v):
    wid = lax.axis_index("s") * _NC + lax.axis_index("c")
    base = wid * _RPW
    pltpu.sync_copy(zmt_hbm, zm_v)

    def chunk_body(c, carry):
        row0 = base + c * _CH
        pltpu.sync_copy(zt_rep_hbm.at[pl.ds(row0, _CH)], zt_v)

        def row_body(i, carry):
            ztv = [zt_v[i, d, :] for d in range(_Z_DIM)]

            for jv in range(_N // _L):
                j0 = jv * _L
                acc = jnp.abs(ztv[0] - zm_v[0, pl.ds(j0, _L)])
                for d in range(1, _Z_DIM):
                    acc = acc + jnp.abs(ztv[d] - zm_v[d, pl.ds(j0, _L)])
                k = acc.astype(jnp.int32).astype(jnp.float32)
                out_v[i, pl.ds(j0, _L)] = k * _A + _B
            return carry

        carry = lax.fori_loop(0, _CH, row_body, carry)
        pltpu.sync_copy(out_v, out_hbm.at[pl.ds(row0, _CH)])
        return carry

    lax.fori_loop(0, _RPW // _CH, chunk_body, 0)


def kernel(zt, ztm1):
    zt_rep = jnp.broadcast_to(zt[:, :, None], (_M, _Z_DIM, _L))
    zmt = ztm1.T

    mesh = plsc.VectorSubcoreMesh(core_axis_name="c", subcore_axis_name="s")
    sc_call = pl.kernel(
        _sc_body,
        mesh=mesh,
        out_type=jax.ShapeDtypeStruct((_M, _N), jnp.float32),
        scratch_types=[
            pltpu.VMEM((_Z_DIM, _N), jnp.float32),
            pltpu.VMEM((_CH, _Z_DIM, _L), jnp.float32),
            pltpu.VMEM((_CH, _N), jnp.float32),
        ],
    )
    return sc_call(zt_rep, zmt)


# speedup vs baseline: 1.7690x; 1.7690x over previous
import functools
import math

import jax
import jax.numpy as jnp
from jax import lax
from jax.experimental import pallas as pl
from jax.experimental.pallas import tpu as pltpu
from jax.experimental.pallas import tpu_sc as plsc

_Z_DIM = 10
_M = 4096
_N = 1024
_NC = 2
_NS = 16
_L = 16
_NW = _NC * _NS
_RPW = _M // _NW
_CH = 16


def _affine_consts():
    p = 0.75
    zs = []
    for k in range(_Z_DIM):
        geo = k * math.log(1.0 - p) + math.log(p)
        log_comb = (
            math.lgamma(_Z_DIM + 1.0)
            - math.lgamma(k + 1.0)
            - math.lgamma(_Z_DIM - k + 1.0)
        )
        zs.append(log_comb + geo)
    mx = max(zs)
    z = mx + math.log(sum(math.exp(v - mx) for v in zs))
    a = math.log(1.0 - p)
    b = math.log(p) - z
    return a, b


_A, _B = _affine_consts()


def _sc_body(zt_rep_hbm, zmt_hbm, out_hbm, zm_v, zt_v, out_v):
    wid = lax.axis_index("s") * _NC + lax.axis_index("c")
    base = wid * _RPW
    pltpu.sync_copy(zmt_hbm, zm_v)

    def chunk_body(c, carry):
        row0 = base + c * _CH
        pltpu.sync_copy(zt_rep_hbm.at[pl.ds(row0, _CH)], zt_v)

        def row_body(i, carry):
            ztv = [zt_v[i, d, :] for d in range(_Z_DIM)]

            def jv_body(jv, carry):
                j0 = jv * _L
                acc = jnp.abs(ztv[0] - zm_v[0, pl.ds(j0, _L)])
                for d in range(1, _Z_DIM):
                    acc = acc + jnp.abs(ztv[d] - zm_v[d, pl.ds(j0, _L)])
                k = acc.astype(jnp.int32).astype(jnp.float32)
                out_v[i, pl.ds(j0, _L)] = k * _A + _B
                return carry

            return lax.fori_loop(0, _N // _L, jv_body, carry, unroll=4)

        carry = lax.fori_loop(0, _CH, row_body, carry)
        pltpu.sync_copy(out_v, out_hbm.at[pl.ds(row0, _CH)])
        return carry

    lax.fori_loop(0, _RPW // _CH, chunk_body, 0)


def kernel(zt, ztm1):
    zt_rep = jnp.broadcast_to(zt[:, :, None], (_M, _Z_DIM, _L))
    zmt = ztm1.T

    mesh = plsc.VectorSubcoreMesh(core_axis_name="c", subcore_axis_name="s")
    sc_call = pl.kernel(
        _sc_body,
        mesh=mesh,
        out_type=jax.ShapeDtypeStruct((_M, _N), jnp.float32),
        scratch_types=[
            pltpu.VMEM((_Z_DIM, _N), jnp.float32),
            pltpu.VMEM((_CH, _Z_DIM, _L), jnp.float32),
            pltpu.VMEM((_CH, _N), jnp.float32),
        ],
    )
    return sc_call(zt_rep, zmt)


# speedup vs baseline: 3.4753x; 1.9646x over previous
import functools
import math

import jax
import jax.numpy as jnp
from jax import lax
from jax.experimental import pallas as pl
from jax.experimental.pallas import tpu as pltpu
from jax.experimental.pallas import tpu_sc as plsc

_Z_DIM = 10
_M = 4096
_N = 1024
_NC = 2
_NS = 16
_L = 16
_NW = _NC * _NS
_RPW = _M // _NW
_CH = 16


def _affine_consts():
    p = 0.75
    zs = []
    for k in range(_Z_DIM):
        geo = k * math.log(1.0 - p) + math.log(p)
        log_comb = (
            math.lgamma(_Z_DIM + 1.0)
            - math.lgamma(k + 1.0)
            - math.lgamma(_Z_DIM - k + 1.0)
        )
        zs.append(log_comb + geo)
    mx = max(zs)
    z = mx + math.log(sum(math.exp(v - mx) for v in zs))
    a = math.log(1.0 - p)
    b = math.log(p) - z
    return a, b


_A, _B = _affine_consts()


def _sc_body(zt_rep_hbm, zmt_hbm, out_hbm, zm_v, zt_v, out_v):
    wid = lax.axis_index("s") * _NC + lax.axis_index("c")
    base = wid * _RPW
    pltpu.sync_copy(zmt_hbm, zm_v)

    def chunk_body(c, carry):
        row0 = base + c * _CH
        pltpu.sync_copy(zt_rep_hbm.at[pl.ds(row0, _CH)], zt_v)

        def row_body(i, carry):
            ztv = [zt_v[i, d, :] for d in range(_Z_DIM)]

            @plsc.parallel_loop(0, _N, step=_L, unroll=4)
            def jv_body(j0):
                acc = jnp.abs(ztv[0] - zm_v[0, pl.ds(j0, _L)])
                for d in range(1, _Z_DIM):
                    acc = acc + jnp.abs(ztv[d] - zm_v[d, pl.ds(j0, _L)])
                k = acc.astype(jnp.int32).astype(jnp.float32)
                out_v[i, pl.ds(j0, _L)] = k * _A + _B

            return carry

        carry = lax.fori_loop(0, _CH, row_body, carry)
        pltpu.sync_copy(out_v, out_hbm.at[pl.ds(row0, _CH)])
        return carry

    lax.fori_loop(0, _RPW // _CH, chunk_body, 0)


def kernel(zt, ztm1):
    zt_rep = jnp.broadcast_to(zt[:, :, None], (_M, _Z_DIM, _L))
    zmt = ztm1.T

    mesh = plsc.VectorSubcoreMesh(core_axis_name="c", subcore_axis_name="s")
    sc_call = pl.kernel(
        _sc_body,
        mesh=mesh,
        out_type=jax.ShapeDtypeStruct((_M, _N), jnp.float32),
        scratch_types=[
            pltpu.VMEM((_Z_DIM, _N), jnp.float32),
            pltpu.VMEM((_CH, _Z_DIM, _L), jnp.float32),
            pltpu.VMEM((_CH, _N), jnp.float32),
        ],
    )
    return sc_call(zt_rep, zmt)


# speedup vs baseline: 3.9120x; 1.1257x over previous
import functools
import math

import jax
import jax.numpy as jnp
from jax import lax
from jax.experimental import pallas as pl
from jax.experimental.pallas import tpu as pltpu
from jax.experimental.pallas import tpu_sc as plsc

_Z_DIM = 10
_M = 4096
_N = 1024
_NC = 2
_NS = 16
_L = 16
_NW = _NC * _NS
_RPW = _M // _NW
_CH = 16


def _affine_consts():
    p = 0.75
    zs = []
    for k in range(_Z_DIM):
        geo = k * math.log(1.0 - p) + math.log(p)
        log_comb = (
            math.lgamma(_Z_DIM + 1.0)
            - math.lgamma(k + 1.0)
            - math.lgamma(_Z_DIM - k + 1.0)
        )
        zs.append(log_comb + geo)
    mx = max(zs)
    z = mx + math.log(sum(math.exp(v - mx) for v in zs))
    a = math.log(1.0 - p)
    b = math.log(p) - z
    return a, b


_A, _B = _affine_consts()


def _sc_body(zt_pack_hbm, zm_pack_hbm, out_hbm, zm_v, zt_v, out_v):
    wid = lax.axis_index("s") * _NC + lax.axis_index("c")
    base = wid * _RPW
    pltpu.sync_copy(zm_pack_hbm, zm_v)

    def chunk_body(c, carry):
        row0 = base + c * _CH
        pltpu.sync_copy(zt_pack_hbm.at[pl.ds(row0, _CH)], zt_v)

        def row_body(i2, carry):
            i0 = i2 * 2
            i1 = i0 + 1
            rows = []
            for i in (i0, i1):
                ztv = [zt_v[i, d, :] for d in range(_Z_DIM)]
                tsa = zt_v[i, _Z_DIM, :]
                rows.append((i, ztv, tsa))

            @plsc.parallel_loop(0, _N, step=_L, unroll=4)
            def jv_body(j0):
                zm = [zm_v[d, pl.ds(j0, _L)] for d in range(_Z_DIM + 1)]
                for i, ztv, tsa in rows:
                    macc = jnp.minimum(ztv[0], zm[0])
                    for d in range(1, _Z_DIM):
                        macc = macc + jnp.minimum(ztv[d], zm[d])
                    dist = (tsa + zm[_Z_DIM]) - macc - macc
                    k = dist.astype(jnp.int32).astype(jnp.float32)
                    out_v[i, pl.ds(j0, _L)] = k * _A + _B

            return carry

        carry = lax.fori_loop(0, _CH // 2, row_body, carry)
        pltpu.sync_copy(out_v, out_hbm.at[pl.ds(row0, _CH)])
        return carry

    lax.fori_loop(0, _RPW // _CH, chunk_body, 0)


def _sc_call(zt, ztm1):
    sa = jnp.sum(zt, axis=1, keepdims=True)
    zt_pack = jnp.broadcast_to(
        jnp.concatenate([zt, sa], axis=1)[:, :, None], (_M, _Z_DIM + 1, _L)
    )
    sb = jnp.sum(ztm1, axis=1, keepdims=True)
    zm_pack = jnp.concatenate([ztm1, sb], axis=1).T

    mesh = plsc.VectorSubcoreMesh(core_axis_name="c", subcore_axis_name="s")
    call = pl.kernel(
        _sc_body,
        mesh=mesh,
        out_type=jax.ShapeDtypeStruct((_M, _N), jnp.float32),
        scratch_types=[
            pltpu.VMEM((_Z_DIM + 1, _N), jnp.float32),
            pltpu.VMEM((_CH, _Z_DIM + 1, _L), jnp.float32),
            pltpu.VMEM((_CH, _N), jnp.float32),
        ],
    )
    return call(zt_pack, zm_pack)


def _tc_kernel(zt_ref, zmt_ref, out_ref):
    sa = jnp.sum(zt_ref[...], axis=1, keepdims=True)
    macc = jnp.minimum(zt_ref[:, 0:1], zmt_ref[0:1, :])
    for d in range(1, _Z_DIM):
        macc = macc + jnp.minimum(zt_ref[:, d : d + 1], zmt_ref[d : d + 1, :])
    t = sa + zmt_ref[_Z_DIM : _Z_DIM + 1, :]
    dist = t - macc - macc
    k = jnp.floor(dist)
    out_ref[...] = k * _A + _B


def _tc_call(zt, ztm1, bi=512):
    m = zt.shape[0]
    sb = jnp.sum(ztm1, axis=1, keepdims=True)
    zm_pack = jnp.concatenate([ztm1, sb], axis=1).T
    return pl.pallas_call(
        _tc_kernel,
        grid=(m // bi,),
        in_specs=[
            pl.BlockSpec((bi, _Z_DIM), lambda i: (i, 0)),
            pl.BlockSpec((_Z_DIM + 1, _N), lambda i: (0, 0)),
        ],
        out_specs=pl.BlockSpec((bi, _N), lambda i: (i, 0)),
        out_shape=jax.ShapeDtypeStruct((m, _N), jnp.float32),
    )(zt, zm_pack)


def kernel(zt, ztm1):
    return _sc_call(zt, ztm1)


# speedup vs baseline: 16.8712x; 4.3127x over previous
import functools
import math

import jax
import jax.numpy as jnp
from jax import lax
from jax.experimental import pallas as pl
from jax.experimental.pallas import tpu as pltpu
from jax.experimental.pallas import tpu_sc as plsc

_Z_DIM = 10
_M = 4096
_N = 1024
_NC = 2
_NS = 16
_L = 16
_NW = _NC * _NS
_RPW = _M // _NW
_CH = 16


def _affine_consts():
    p = 0.75
    zs = []
    for k in range(_Z_DIM):
        geo = k * math.log(1.0 - p) + math.log(p)
        log_comb = (
            math.lgamma(_Z_DIM + 1.0)
            - math.lgamma(k + 1.0)
            - math.lgamma(_Z_DIM - k + 1.0)
        )
        zs.append(log_comb + geo)
    mx = max(zs)
    z = mx + math.log(sum(math.exp(v - mx) for v in zs))
    a = math.log(1.0 - p)
    b = math.log(p) - z
    return a, b


_A, _B = _affine_consts()


def _sc_body(zt_pack_hbm, zm_pack_hbm, out_hbm, zm_v, zt_v, out_v):
    wid = lax.axis_index("s") * _NC + lax.axis_index("c")
    base = wid * _RPW
    pltpu.sync_copy(zm_pack_hbm, zm_v)

    def chunk_body(c, carry):
        row0 = base + c * _CH
        pltpu.sync_copy(zt_pack_hbm.at[pl.ds(row0, _CH)], zt_v)

        def row_body(i2, carry):
            i0 = i2 * 2
            i1 = i0 + 1
            rows = []
            for i in (i0, i1):
                ztv = [zt_v[i, d, :] for d in range(_Z_DIM)]
                tsa = zt_v[i, _Z_DIM, :]
                rows.append((i, ztv, tsa))

            @plsc.parallel_loop(0, _N, step=_L, unroll=4)
            def jv_body(j0):
                zm = [zm_v[d, pl.ds(j0, _L)] for d in range(_Z_DIM + 1)]
                for i, ztv, tsa in rows:
                    macc = jnp.minimum(ztv[0], zm[0])
                    for d in range(1, _Z_DIM):
                        macc = macc + jnp.minimum(ztv[d], zm[d])
                    dist = (tsa + zm[_Z_DIM]) - macc - macc
                    k = dist.astype(jnp.int32).astype(jnp.float32)
                    out_v[i, pl.ds(j0, _L)] = k * _A + _B

            return carry

        carry = lax.fori_loop(0, _CH // 2, row_body, carry)
        pltpu.sync_copy(out_v, out_hbm.at[pl.ds(row0, _CH)])
        return carry

    lax.fori_loop(0, _RPW // _CH, chunk_body, 0)


def _sc_call(zt, ztm1):
    sa = jnp.sum(zt, axis=1, keepdims=True)
    zt_pack = jnp.broadcast_to(
        jnp.concatenate([zt, sa], axis=1)[:, :, None], (_M, _Z_DIM + 1, _L)
    )
    sb = jnp.sum(ztm1, axis=1, keepdims=True)
    zm_pack = jnp.concatenate([ztm1, sb], axis=1).T

    mesh = plsc.VectorSubcoreMesh(core_axis_name="c", subcore_axis_name="s")
    call = pl.kernel(
        _sc_body,
        mesh=mesh,
        out_type=jax.ShapeDtypeStruct((_M, _N), jnp.float32),
        scratch_types=[
            pltpu.VMEM((_Z_DIM + 1, _N), jnp.float32),
            pltpu.VMEM((_CH, _Z_DIM + 1, _L), jnp.float32),
            pltpu.VMEM((_CH, _N), jnp.float32),
        ],
    )
    return call(zt_pack, zm_pack)


def _tc_kernel(zt_ref, zmt_ref, out_ref):
    sa = jnp.sum(zt_ref[...], axis=1, keepdims=True)
    macc = jnp.minimum(zt_ref[:, 0:1], zmt_ref[0:1, :])
    for d in range(1, _Z_DIM):
        macc = macc + jnp.minimum(zt_ref[:, d : d + 1], zmt_ref[d : d + 1, :])
    t = sa + zmt_ref[_Z_DIM : _Z_DIM + 1, :]
    dist = t - macc - macc
    k = jnp.floor(dist)
    out_ref[...] = k * _A + _B


def _tc_call(zt, ztm1, bi=512):
    m = zt.shape[0]
    sb = jnp.sum(ztm1, axis=1, keepdims=True)
    zm_pack = jnp.concatenate([ztm1, sb], axis=1).T
    return pl.pallas_call(
        _tc_kernel,
        grid=(m // bi,),
        in_specs=[
            pl.BlockSpec((bi, _Z_DIM), lambda i: (i, 0)),
            pl.BlockSpec((_Z_DIM + 1, _N), lambda i: (0, 0)),
        ],
        out_specs=pl.BlockSpec((bi, _N), lambda i: (i, 0)),
        out_shape=jax.ShapeDtypeStruct((m, _N), jnp.float32),
    )(zt, zm_pack)


def kernel(zt, ztm1):
    return _tc_call(zt, ztm1)


# speedup vs baseline: 18.3970x; 1.0904x over previous
import functools
import math

import jax
import jax.numpy as jnp
from jax import lax
from jax.experimental import pallas as pl
from jax.experimental.pallas import tpu as pltpu
from jax.experimental.pallas import tpu_sc as plsc

_Z_DIM = 10
_M = 4096
_N = 1024
_NC = 2
_NS = 16
_L = 16
_NW = _NC * _NS
_RPW = _M // _NW
_CH = 16


def _affine_consts():
    p = 0.75
    zs = []
    for k in range(_Z_DIM):
        geo = k * math.log(1.0 - p) + math.log(p)
        log_comb = (
            math.lgamma(_Z_DIM + 1.0)
            - math.lgamma(k + 1.0)
            - math.lgamma(_Z_DIM - k + 1.0)
        )
        zs.append(log_comb + geo)
    mx = max(zs)
    z = mx + math.log(sum(math.exp(v - mx) for v in zs))
    a = math.log(1.0 - p)
    b = math.log(p) - z
    return a, b


_A, _B = _affine_consts()


def _sc_body(zt_pack_hbm, zm_pack_hbm, out_hbm, zm_v, zt_v, out_v):
    wid = lax.axis_index("s") * _NC + lax.axis_index("c")
    base = wid * _RPW
    pltpu.sync_copy(zm_pack_hbm, zm_v)

    def chunk_body(c, carry):
        row0 = base + c * _CH
        pltpu.sync_copy(zt_pack_hbm.at[pl.ds(row0, _CH)], zt_v)

        def row_body(i2, carry):
            i0 = i2 * 2
            i1 = i0 + 1
            rows = []
            for i in (i0, i1):
                ztv = [zt_v[i, d, :] for d in range(_Z_DIM)]
                tsa = zt_v[i, _Z_DIM, :]
                rows.append((i, ztv, tsa))

            @plsc.parallel_loop(0, _N, step=_L, unroll=4)
            def jv_body(j0):
                zm = [zm_v[d, pl.ds(j0, _L)] for d in range(_Z_DIM + 1)]
                for i, ztv, tsa in rows:
                    macc = jnp.minimum(ztv[0], zm[0])
                    for d in range(1, _Z_DIM):
                        macc = macc + jnp.minimum(ztv[d], zm[d])
                    dist = (tsa + zm[_Z_DIM]) - macc - macc
                    k = dist.astype(jnp.int32).astype(jnp.float32)
                    out_v[i, pl.ds(j0, _L)] = k * _A + _B

            return carry

        carry = lax.fori_loop(0, _CH // 2, row_body, carry)
        pltpu.sync_copy(out_v, out_hbm.at[pl.ds(row0, _CH)])
        return carry

    lax.fori_loop(0, _RPW // _CH, chunk_body, 0)


def _sc_call(zt, ztm1):
    sa = jnp.sum(zt, axis=1, keepdims=True)
    zt_pack = jnp.broadcast_to(
        jnp.concatenate([zt, sa], axis=1)[:, :, None], (_M, _Z_DIM + 1, _L)
    )
    sb = jnp.sum(ztm1, axis=1, keepdims=True)
    zm_pack = jnp.concatenate([ztm1, sb], axis=1).T

    mesh = plsc.VectorSubcoreMesh(core_axis_name="c", subcore_axis_name="s")
    call = pl.kernel(
        _sc_body,
        mesh=mesh,
        out_type=jax.ShapeDtypeStruct((_M, _N), jnp.float32),
        scratch_types=[
            pltpu.VMEM((_Z_DIM + 1, _N), jnp.float32),
            pltpu.VMEM((_CH, _Z_DIM + 1, _L), jnp.float32),
            pltpu.VMEM((_CH, _N), jnp.float32),
        ],
    )
    return call(zt_pack, zm_pack)


def _tc_kernel(zt_ref, zmt_ref, out_ref):
    sa = jnp.sum(zt_ref[...], axis=1, keepdims=True)
    sb = zmt_ref[0:1, :]
    for d in range(1, _Z_DIM):
        sb = sb + zmt_ref[d : d + 1, :]
    macc = jnp.minimum(zt_ref[:, 0:1], zmt_ref[0:1, :])
    for d in range(1, _Z_DIM):
        macc = macc + jnp.minimum(zt_ref[:, d : d + 1], zmt_ref[d : d + 1, :])
    dist = (sa + sb) - macc - macc
    k = jnp.floor(dist)
    out_ref[...] = k * _A + _B


def _tc_call(zt, ztm1, bi=512):
    m = zt.shape[0]
    zmt = ztm1.T
    return pl.pallas_call(
        _tc_kernel,
        grid=(m // bi,),
        in_specs=[
            pl.BlockSpec((bi, _Z_DIM), lambda i: (i, 0)),
            pl.BlockSpec((_Z_DIM, _N), lambda i: (0, 0)),
        ],
        out_specs=pl.BlockSpec((bi, _N), lambda i: (i, 0)),
        out_shape=jax.ShapeDtypeStruct((m, _N), jnp.float32),
    )(zt, zmt)


def kernel(zt, ztm1):
    return _tc_call(zt, ztm1)
